# initial kernel scaffold (unmeasured)
import jax
import jax.numpy as jnp
from jax import lax
from jax.experimental import pallas as pl
from jax.experimental.pallas import tpu as pltpu


def kernel(
    x,
):
    def body(*refs):
        pass

    out_shape = jax.ShapeDtypeStruct(..., jnp.float32)
    return pl.pallas_call(body, out_shape=out_shape)(...)


# baseline (device time: 11583 ns/iter reference)
import jax
import jax.numpy as jnp
from jax import lax
from jax.experimental import pallas as pl
from jax.experimental.pallas import tpu as pltpu

N_DEV = 8


def kernel(x):
    m, n = x.shape

    def body(x_ref, o_ref, halo_ref, send_sems, recv_sems):
        my = lax.axis_index("i")

        @pl.when(my < N_DEV - 1)
        def _():
            rdma = pltpu.make_async_remote_copy(
                src_ref=x_ref.at[pl.ds(m - 1, 1)],
                dst_ref=halo_ref.at[0],
                send_sem=send_sems.at[0],
                recv_sem=recv_sems.at[0],
                device_id=(my + 1,),
                device_id_type=pl.DeviceIdType.MESH,
            )
            rdma.start()

        @pl.when(my > 0)
        def _():
            rdma = pltpu.make_async_remote_copy(
                src_ref=x_ref.at[pl.ds(0, 1)],
                dst_ref=halo_ref.at[1],
                send_sem=send_sems.at[1],
                recv_sem=recv_sems.at[1],
                device_id=(my - 1,),
                device_id_type=pl.DeviceIdType.MESH,
            )
            rdma.start()

        o_ref[1 : m - 1, :] = (
            0.25 * x_ref[0 : m - 2, :]
            + 0.5 * x_ref[1 : m - 1, :]
            + 0.25 * x_ref[2:m, :]
        )

        @pl.when(my == 0)
        def _():
            o_ref[0:1, :] = x_ref[0:1, :]

        @pl.when(my > 0)
        def _():
            recv = pltpu.make_async_remote_copy(
                src_ref=x_ref.at[pl.ds(0, 1)],
                dst_ref=halo_ref.at[0],
                send_sem=send_sems.at[1],
                recv_sem=recv_sems.at[0],
                device_id=(my - 1,),
                device_id_type=pl.DeviceIdType.MESH,
            )
            recv.wait_recv()
            o_ref[0:1, :] = (
                0.25 * halo_ref[0, :, :]
                + 0.5 * x_ref[0:1, :]
                + 0.25 * x_ref[1:2, :]
            )

        @pl.when(my == N_DEV - 1)
        def _():
            o_ref[m - 1 : m, :] = x_ref[m - 1 : m, :]

        @pl.when(my < N_DEV - 1)
        def _():
            recv = pltpu.make_async_remote_copy(
                src_ref=x_ref.at[pl.ds(m - 1, 1)],
                dst_ref=halo_ref.at[1],
                send_sem=send_sems.at[0],
                recv_sem=recv_sems.at[1],
                device_id=(my + 1,),
                device_id_type=pl.DeviceIdType.MESH,
            )
            recv.wait_recv()
            o_ref[m - 1 : m, :] = (
                0.25 * x_ref[m - 2 : m - 1, :]
                + 0.5 * x_ref[m - 1 : m, :]
                + 0.25 * halo_ref[1, :, :]
            )

        @pl.when(my < N_DEV - 1)
        def _():
            send = pltpu.make_async_remote_copy(
                src_ref=x_ref.at[pl.ds(m - 1, 1)],
                dst_ref=halo_ref.at[0],
                send_sem=send_sems.at[0],
                recv_sem=recv_sems.at[0],
                device_id=(my + 1,),
                device_id_type=pl.DeviceIdType.MESH,
            )
            send.wait_send()

        @pl.when(my > 0)
        def _():
            send = pltpu.make_async_remote_copy(
                src_ref=x_ref.at[pl.ds(0, 1)],
                dst_ref=halo_ref.at[1],
                send_sem=send_sems.at[1],
                recv_sem=recv_sems.at[1],
                device_id=(my - 1,),
                device_id_type=pl.DeviceIdType.MESH,
            )
            send.wait_send()

    return pl.pallas_call(
        body,
        out_shape=jax.ShapeDtypeStruct((m, n), x.dtype),
        in_specs=[pl.BlockSpec(memory_space=pltpu.VMEM)],
        out_specs=pl.BlockSpec(memory_space=pltpu.VMEM),
        scratch_shapes=[
            pltpu.VMEM((2, 1, n), x.dtype),
            pltpu.SemaphoreType.DMA((2,)),
            pltpu.SemaphoreType.DMA((2,)),
        ],
    )(x)


# device time: 7987 ns/iter; 1.4502x vs baseline; 1.4502x over previous
import jax
import jax.numpy as jnp
from jax import lax
from jax.experimental import pallas as pl
from jax.experimental.pallas import tpu as pltpu

N_DEV = 8


def kernel(x):
    m, n = x.shape

    def body(x_ref, o_ref, halo_ref, send_sems, recv_sems):
        my = lax.axis_index("i")

        barrier_sem = pltpu.get_barrier_semaphore()

        @pl.when(my > 0)
        def _():
            pl.semaphore_signal(
                barrier_sem, inc=1,
                device_id=(my - 1,), device_id_type=pl.DeviceIdType.MESH,
            )

        @pl.when(my < N_DEV - 1)
        def _():
            pl.semaphore_signal(
                barrier_sem, inc=1,
                device_id=(my + 1,), device_id_type=pl.DeviceIdType.MESH,
            )

        n_neighbors = jnp.where((my == 0) | (my == N_DEV - 1), 1, 2)
        pl.semaphore_wait(barrier_sem, n_neighbors)

        @pl.when(my < N_DEV - 1)
        def _():
            rdma = pltpu.make_async_remote_copy(
                src_ref=x_ref.at[pl.ds(m - 1, 1)],
                dst_ref=halo_ref.at[0],
                send_sem=send_sems.at[0],
                recv_sem=recv_sems.at[0],
                device_id=(my + 1,),
                device_id_type=pl.DeviceIdType.MESH,
            )
            rdma.start()

        @pl.when(my > 0)
        def _():
            rdma = pltpu.make_async_remote_copy(
                src_ref=x_ref.at[pl.ds(0, 1)],
                dst_ref=halo_ref.at[1],
                send_sem=send_sems.at[1],
                recv_sem=recv_sems.at[1],
                device_id=(my - 1,),
                device_id_type=pl.DeviceIdType.MESH,
            )
            rdma.start()

        o_ref[1 : m - 1, :] = (
            0.25 * x_ref[0 : m - 2, :]
            + 0.5 * x_ref[1 : m - 1, :]
            + 0.25 * x_ref[2:m, :]
        )

        @pl.when(my == 0)
        def _():
            o_ref[0:1, :] = x_ref[0:1, :]

        @pl.when(my > 0)
        def _():
            recv = pltpu.make_async_remote_copy(
                src_ref=x_ref.at[pl.ds(0, 1)],
                dst_ref=halo_ref.at[0],
                send_sem=send_sems.at[1],
                recv_sem=recv_sems.at[0],
                device_id=(my - 1,),
                device_id_type=pl.DeviceIdType.MESH,
            )
            recv.wait_recv()
            o_ref[0:1, :] = (
                0.25 * halo_ref[0, :, :]
                + 0.5 * x_ref[0:1, :]
                + 0.25 * x_ref[1:2, :]
            )

        @pl.when(my == N_DEV - 1)
        def _():
            o_ref[m - 1 : m, :] = x_ref[m - 1 : m, :]

        @pl.when(my < N_DEV - 1)
        def _():
            recv = pltpu.make_async_remote_copy(
                src_ref=x_ref.at[pl.ds(m - 1, 1)],
                dst_ref=halo_ref.at[1],
                send_sem=send_sems.at[0],
                recv_sem=recv_sems.at[1],
                device_id=(my + 1,),
                device_id_type=pl.DeviceIdType.MESH,
            )
            recv.wait_recv()
            o_ref[m - 1 : m, :] = (
                0.25 * x_ref[m - 2 : m - 1, :]
                + 0.5 * x_ref[m - 1 : m, :]
                + 0.25 * halo_ref[1, :, :]
            )

        @pl.when(my < N_DEV - 1)
        def _():
            send = pltpu.make_async_remote_copy(
                src_ref=x_ref.at[pl.ds(m - 1, 1)],
                dst_ref=halo_ref.at[0],
                send_sem=send_sems.at[0],
                recv_sem=recv_sems.at[0],
                device_id=(my + 1,),
                device_id_type=pl.DeviceIdType.MESH,
            )
            send.wait_send()

        @pl.when(my > 0)
        def _():
            send = pltpu.make_async_remote_copy(
                src_ref=x_ref.at[pl.ds(0, 1)],
                dst_ref=halo_ref.at[1],
                send_sem=send_sems.at[1],
                recv_sem=recv_sems.at[1],
                device_id=(my - 1,),
                device_id_type=pl.DeviceIdType.MESH,
            )
            send.wait_send()

    return pl.pallas_call(
        body,
        out_shape=jax.ShapeDtypeStruct((m, n), x.dtype),
        in_specs=[pl.BlockSpec(memory_space=pltpu.VMEM)],
        out_specs=pl.BlockSpec(memory_space=pltpu.VMEM),
        scratch_shapes=[
            pltpu.VMEM((2, 1, n), x.dtype),
            pltpu.SemaphoreType.DMA((2,)),
            pltpu.SemaphoreType.DMA((2,)),
        ],
        compiler_params=pltpu.CompilerParams(collective_id=0),
    )(x)
